# baseline (speedup 1.0000x reference)
import functools

import jax
import jax.numpy as jnp
from jax import lax
from jax.experimental import pallas as pl
from jax.experimental.pallas import tpu as pltpu
from jax.experimental.pallas import tpu_sc as plsc

NC = 2
NS = 16
LANES = 16
NW = NC * NS

CH = 40
NP = 8


def _sc_gather_mean(input_ids, embed_table, B, S, D):
    BPW = B // NW
    DV = D // LANES
    NCH = S // CH

    mesh = plsc.VectorSubcoreMesh(core_axis_name="c", subcore_axis_name="s")

    @functools.partial(
        pl.kernel,
        out_type=jax.ShapeDtypeStruct((B, D), jnp.float32),
        mesh=mesh,
        scratch_types=[
            pltpu.VMEM((BPW * S,), jnp.int32),
            pltpu.VMEM((S, D), jnp.float32),
            pltpu.VMEM((S, D), jnp.float32),
            pltpu.VMEM((BPW, D), jnp.float32),
            pltpu.SemaphoreType.DMA,
            pltpu.SemaphoreType.DMA,
        ],
        compiler_params=pltpu.CompilerParams(use_tc_tiling_on_sc=False),
    )
    def sc_fn(idx_hbm, table_hbm, out_hbm, idx_v, buf_a, buf_b, out_v,
              sem_a, sem_b):
        wid = lax.axis_index("s") * NC + lax.axis_index("c")
        base = wid * BPW
        pltpu.sync_copy(idx_hbm.at[pl.ds(base * S, BPW * S)], idx_v)
        scale = jnp.float32(1.0 / S)

        def issue_row(r, buf, sem):
            for c in range(NCH):
                pltpu.async_copy(
                    table_hbm.at[idx_v.at[pl.ds(r * S + c * CH, CH)]],
                    buf.at[pl.ds(c * CH, CH)],
                    sem,
                )

        def drain_row(buf, sem):
            pltpu.make_async_copy(table_hbm.at[pl.ds(0, S)], buf, sem).wait()

        def compute_row(r, buf):
            for d in range(DV):
                p = [jnp.zeros((LANES,), jnp.float32) for _ in range(NP)]
                for j in range(S):
                    p[j % NP] = p[j % NP] + buf[j, pl.ds(d * LANES, LANES)]
                while len(p) > 1:
                    p = [p[i] + p[i + 1] for i in range(0, len(p), 2)]
                out_v[r, pl.ds(d * LANES, LANES)] = p[0] * scale

        issue_row(0, buf_a, sem_a)
        issue_row(1, buf_b, sem_b)

        def pair_fn(i, carry):
            r0 = 2 * i
            drain_row(buf_a, sem_a)
            compute_row(r0, buf_a)

            @pl.when(r0 + 2 < BPW)
            def _():
                issue_row(r0 + 2, buf_a, sem_a)

            drain_row(buf_b, sem_b)
            compute_row(r0 + 1, buf_b)

            @pl.when(r0 + 3 < BPW)
            def _():
                issue_row(r0 + 3, buf_b, sem_b)

            return carry

        lax.fori_loop(0, BPW // 2, pair_fn, 0)
        pltpu.sync_copy(out_v, out_hbm.at[pl.ds(base, BPW)])

    return sc_fn(input_ids, embed_table)


def _sc_transpose_pad(tt, tail_pad, V, D):
    CB = 128
    NF = V // CB
    TAIL = V - NF * CB
    NIT = (NF + NW - 1) // NW
    DV = D // LANES

    mesh = plsc.VectorSubcoreMesh(core_axis_name="c", subcore_axis_name="s")

    @functools.partial(
        pl.kernel,
        out_type=jax.ShapeDtypeStruct((V, 128), jnp.float32),
        mesh=mesh,
        scratch_types=[
            pltpu.VMEM((D, CB), jnp.float32),
            pltpu.VMEM((D, CB), jnp.float32),
            pltpu.VMEM((CB, 128), jnp.float32),
            pltpu.VMEM((CB, 128), jnp.float32),
            pltpu.SemaphoreType.DMA,
            pltpu.SemaphoreType.DMA,
            pltpu.SemaphoreType.DMA,
            pltpu.SemaphoreType.DMA,
        ],
        compiler_params=pltpu.CompilerParams(
            use_tc_tiling_on_sc=True, needs_layout_passes=False
        ),
    )
    def tr_fn(tt_hbm, tail_hbm, out_hbm, in_a, in_b, ot_a, ot_b, sia, sib,
              soa, sob):
        wid = lax.axis_index("s") * NC + lax.axis_index("c")
        iota16 = jax.lax.iota(jnp.int32, LANES)
        rows = [iota16 + d * LANES for d in range(DV)]

        def zinit(r, carry):
            for ob in (ot_a, ot_b):
                for d in range(DV):
                    ob[r, pl.ds(D + d * LANES, LANES)] = jnp.zeros(
                        (LANES,), jnp.float32
                    )
            return carry

        lax.fori_loop(0, CB, zinit, 0)

        def chunk_of(i):
            return wid + NW * i

        def issue_read(i, buf, sem):
            c = jnp.minimum(chunk_of(i), NF - 1)
            pltpu.async_copy(
                tt_hbm.at[pl.ds(0, D), pl.ds(c * CB, CB)], buf, sem
            )

        def drain_read(buf, sem):
            pltpu.make_async_copy(
                tt_hbm.at[pl.ds(0, D), pl.ds(0, CB)], buf, sem
            ).wait()

        def transpose(inb, outb, nv):
            for v in range(nv):
                colv = jnp.zeros((LANES,), jnp.int32) + v
                for d in range(DV):
                    outb[v, pl.ds(d * LANES, LANES)] = plsc.load_gather(
                        inb, [rows[d], colv]
                    )

        def issue_write(i, outb, sem):
            c = chunk_of(i)

            @pl.when(c < NF)
            def _():
                pltpu.async_copy(
                    outb, out_hbm.at[pl.ds(c * CB, CB)], sem
                )

        def drain_write(i, outb, sem):
            c = chunk_of(i)

            @pl.when(c < NF)
            def _():
                pltpu.make_async_copy(
                    outb, out_hbm.at[pl.ds(0, CB)], sem
                ).wait()

        issue_read(0, in_a, sia)
        issue_read(1, in_b, sib)

        def pair_fn(p, carry):
            for (ie, inb, outb, si, so) in (
                (2 * p, in_a, ot_a, sia, soa),
                (2 * p + 1, in_b, ot_b, sib, sob),
            ):
                drain_read(inb, si)

                @pl.when(ie >= 2)
                def _():
                    drain_write(ie - 2, outb, so)

                transpose(inb, outb, CB)
                issue_write(ie, outb, so)

                @pl.when(ie + 2 < NIT)
                def _():
                    issue_read(ie + 2, inb, si)

            return carry

        lax.fori_loop(0, NIT // 2, pair_fn, 0)
        if NIT % 2:
            ie = NIT - 1
            drain_read(in_a, sia)

            @pl.when(ie >= 2)
            def _():
                drain_write(ie - 2, ot_a, soa)

            transpose(in_a, ot_a, CB)
            issue_write(ie, ot_a, soa)

        la = NIT - 1 if NIT % 2 else NIT - 2
        drain_write(la, ot_a, soa)
        drain_write(NIT - 2 if NIT % 2 else NIT - 1, ot_b, sob)

        if TAIL:

            @pl.when(wid == NW - 1)
            def _():
                pltpu.sync_copy(tail_hbm, ot_a.at[pl.ds(0, TAIL)])
                pltpu.sync_copy(
                    ot_a.at[pl.ds(0, TAIL)], out_hbm.at[pl.ds(NF * CB, TAIL)]
                )

    return tr_fn(tt, tail_pad)


def _tc_linear_logsoftmax(x, W, b2, B, D, L):
    BT = 512

    def tc_body(x_ref, w_ref, b_ref, o_ref):
        logits = (
            jnp.dot(x_ref[...], w_ref[...], preferred_element_type=jnp.float32)
            + b_ref[...]
        )
        m = jnp.max(logits, axis=-1, keepdims=True)
        e = jnp.exp(logits - m)
        lse = jnp.log(jnp.sum(e, axis=-1, keepdims=True)) + m
        o_ref[...] = logits - lse

    return pl.pallas_call(
        tc_body,
        grid=(B // BT,),
        in_specs=[
            pl.BlockSpec((BT, D), lambda i: (i, 0)),
            pl.BlockSpec((D, L), lambda i: (0, 0)),
            pl.BlockSpec((1, L), lambda i: (0, 0)),
        ],
        out_specs=pl.BlockSpec((BT, L), lambda i: (i, 0)),
        out_shape=jax.ShapeDtypeStruct((B, L), jnp.float32),
    )(x, W, b2)


def kernel(input_ids, seq_len, embed_table, W, b):
    del seq_len
    B, S = input_ids.shape
    V, D = embed_table.shape
    L = W.shape[1]

    ntail = V % 128
    tail_pad = jnp.pad(
        embed_table[V - ntail :], ((0, 0), (0, 128 - D))
    )
    table_pad = _sc_transpose_pad(embed_table.T, tail_pad, V, D).reshape(
        2 * V, D
    )
    idx_flat = input_ids.astype(jnp.int32).reshape(B * S) * 2
    pooled = _sc_gather_mean(idx_flat, table_pad, B, S, D)
    return _tc_linear_logsoftmax(pooled, W, b.reshape(1, L), B, D, L)


# speedup vs baseline: 1.1195x; 1.1195x over previous
import functools

import jax
import jax.numpy as jnp
from jax import lax
from jax.experimental import pallas as pl
from jax.experimental.pallas import tpu as pltpu
from jax.experimental.pallas import tpu_sc as plsc

NC = 2
NS = 16
LANES = 16
NW = NC * NS

CH = 40
NP = 8


def _sc_gather_mean(input_ids, embed_table, B, S, D):
    BPW = B // NW
    DV = D // LANES
    NCH = S // CH

    mesh = plsc.VectorSubcoreMesh(core_axis_name="c", subcore_axis_name="s")

    @functools.partial(
        pl.kernel,
        out_type=jax.ShapeDtypeStruct((B, D), jnp.float32),
        mesh=mesh,
        scratch_types=[
            pltpu.VMEM((BPW * S,), jnp.int32),
            pltpu.VMEM((S, D), jnp.float32),
            pltpu.VMEM((S, D), jnp.float32),
            pltpu.VMEM((BPW, D), jnp.float32),
            pltpu.SemaphoreType.DMA,
            pltpu.SemaphoreType.DMA,
        ],
        compiler_params=pltpu.CompilerParams(use_tc_tiling_on_sc=False),
    )
    def sc_fn(idx_hbm, table_hbm, out_hbm, idx_v, buf_a, buf_b, out_v,
              sem_a, sem_b):
        wid = lax.axis_index("s") * NC + lax.axis_index("c")
        base = wid * BPW
        pltpu.sync_copy(idx_hbm.at[pl.ds(base * S, BPW * S)], idx_v)
        scale = jnp.float32(1.0 / S)

        def issue_row(r, buf, sem):
            for c in range(NCH):
                pltpu.async_copy(
                    table_hbm.at[idx_v.at[pl.ds(r * S + c * CH, CH)]],
                    buf.at[pl.ds(c * CH, CH)],
                    sem,
                )

        def drain_row(buf, sem):
            pltpu.make_async_copy(table_hbm.at[pl.ds(0, S)], buf, sem).wait()

        def compute_row(r, buf):
            for d in range(DV):
                p = [jnp.zeros((LANES,), jnp.float32) for _ in range(NP)]
                for j in range(S):
                    p[j % NP] = p[j % NP] + buf[j, pl.ds(d * LANES, LANES)]
                while len(p) > 1:
                    p = [p[i] + p[i + 1] for i in range(0, len(p), 2)]
                out_v[r, pl.ds(d * LANES, LANES)] = p[0] * scale

        issue_row(0, buf_a, sem_a)
        issue_row(1, buf_b, sem_b)

        def pair_fn(i, carry):
            r0 = 2 * i
            drain_row(buf_a, sem_a)
            compute_row(r0, buf_a)

            @pl.when(r0 + 2 < BPW)
            def _():
                issue_row(r0 + 2, buf_a, sem_a)

            drain_row(buf_b, sem_b)
            compute_row(r0 + 1, buf_b)

            @pl.when(r0 + 3 < BPW)
            def _():
                issue_row(r0 + 3, buf_b, sem_b)

            return carry

        lax.fori_loop(0, BPW // 2, pair_fn, 0)
        pltpu.sync_copy(out_v, out_hbm.at[pl.ds(base, BPW)])

    return sc_fn(input_ids, embed_table)


def _sc_transpose_pad(tt, tail_pad, V, D):
    CB = 128
    NF = V // CB
    TAIL = V - NF * CB
    NIT = (NF + NW - 1) // NW
    DV = D // LANES

    mesh = plsc.VectorSubcoreMesh(core_axis_name="c", subcore_axis_name="s")

    @functools.partial(
        pl.kernel,
        out_type=jax.ShapeDtypeStruct((V, 128), jnp.float32),
        mesh=mesh,
        scratch_types=[
            pltpu.VMEM((D, CB), jnp.float32),
            pltpu.VMEM((D, CB), jnp.float32),
            pltpu.VMEM((CB, 128), jnp.float32),
            pltpu.VMEM((CB, 128), jnp.float32),
            pltpu.SemaphoreType.DMA,
            pltpu.SemaphoreType.DMA,
            pltpu.SemaphoreType.DMA,
            pltpu.SemaphoreType.DMA,
        ],
        compiler_params=pltpu.CompilerParams(
            use_tc_tiling_on_sc=True, needs_layout_passes=False
        ),
    )
    def tr_fn(tt_hbm, tail_hbm, out_hbm, in_a, in_b, ot_a, ot_b, sia, sib,
              soa, sob):
        wid = lax.axis_index("s") * NC + lax.axis_index("c")
        iota16 = jax.lax.iota(jnp.int32, LANES)
        rows = [iota16 + d * LANES for d in range(DV)]

        def chunk_of(i):
            return wid + NW * i

        def issue_read(i, buf, sem):
            c = jnp.minimum(chunk_of(i), NF - 1)
            pltpu.async_copy(
                tt_hbm.at[pl.ds(0, D), pl.ds(c * CB, CB)], buf, sem
            )

        def drain_read(buf, sem):
            pltpu.make_async_copy(
                tt_hbm.at[pl.ds(0, D), pl.ds(0, CB)], buf, sem
            ).wait()

        def transpose(inb, outb):
            def tv(q, carry):
                v0 = q * 8
                for kv in range(8):
                    v = v0 + kv
                    colv = jnp.zeros((LANES,), jnp.int32) + v
                    for d in range(DV):
                        outb[v, pl.ds(d * LANES, LANES)] = plsc.load_gather(
                            inb, [rows[d], colv]
                        )
                return carry

            lax.fori_loop(0, CB // 8, tv, 0)

        def issue_write(i, outb, sem):
            c = chunk_of(i)

            @pl.when(c < NF)
            def _():
                pltpu.async_copy(outb, out_hbm.at[pl.ds(c * CB, CB)], sem)

        def drain_write(i, outb, sem):
            c = chunk_of(i)

            @pl.when(c < NF)
            def _():
                pltpu.make_async_copy(
                    outb, out_hbm.at[pl.ds(0, CB)], sem
                ).wait()

        issue_read(0, in_a, sia)
        issue_read(1, in_b, sib)

        def pair_fn(p, carry):
            for (ie, inb, outb, si, so) in (
                (2 * p, in_a, ot_a, sia, soa),
                (2 * p + 1, in_b, ot_b, sib, sob),
            ):
                drain_read(inb, si)

                @pl.when(ie >= 2)
                def _():
                    drain_write(ie - 2, outb, so)

                transpose(inb, outb)
                issue_write(ie, outb, so)

                @pl.when(ie + 2 < NIT)
                def _():
                    issue_read(ie + 2, inb, si)

            return carry

        lax.fori_loop(0, NIT // 2, pair_fn, 0)
        if NIT % 2:
            ie = NIT - 1
            drain_read(in_a, sia)

            @pl.when(ie >= 2)
            def _():
                drain_write(ie - 2, ot_a, soa)

            transpose(in_a, ot_a)
            issue_write(ie, ot_a, soa)

        drain_write(NIT - 1 if NIT % 2 else NIT - 2, ot_a, soa)
        drain_write(NIT - 2 if NIT % 2 else NIT - 1, ot_b, sob)

        if TAIL:

            @pl.when(wid == NW - 1)
            def _():
                pltpu.sync_copy(tail_hbm, ot_a.at[pl.ds(0, TAIL)])
                pltpu.sync_copy(
                    ot_a.at[pl.ds(0, TAIL)], out_hbm.at[pl.ds(NF * CB, TAIL)]
                )

    return tr_fn(tt, tail_pad)


def _tc_linear_logsoftmax(x, W, b2, B, D, L):
    BT = 512

    def tc_body(x_ref, w_ref, b_ref, o_ref):
        logits = (
            jnp.dot(x_ref[...], w_ref[...], preferred_element_type=jnp.float32)
            + b_ref[...]
        )
        m = jnp.max(logits, axis=-1, keepdims=True)
        e = jnp.exp(logits - m)
        lse = jnp.log(jnp.sum(e, axis=-1, keepdims=True)) + m
        o_ref[...] = logits - lse

    return pl.pallas_call(
        tc_body,
        grid=(B // BT,),
        in_specs=[
            pl.BlockSpec((BT, D), lambda i: (i, 0)),
            pl.BlockSpec((D, L), lambda i: (0, 0)),
            pl.BlockSpec((1, L), lambda i: (0, 0)),
        ],
        out_specs=pl.BlockSpec((BT, L), lambda i: (i, 0)),
        out_shape=jax.ShapeDtypeStruct((B, L), jnp.float32),
    )(x, W, b2)


def kernel(input_ids, seq_len, embed_table, W, b):
    del seq_len
    B, S = input_ids.shape
    V, D = embed_table.shape
    L = W.shape[1]

    ntail = V % 128
    tail_pad = jnp.pad(embed_table[V - ntail :], ((0, 0), (0, 128 - D)))
    table_pad = _sc_transpose_pad(embed_table.T, tail_pad, V, D).reshape(
        2 * V, D
    )
    idx_flat = input_ids.astype(jnp.int32).reshape(B * S) * 2
    pooled = _sc_gather_mean(idx_flat, table_pad, B, S, D)
    return _tc_linear_logsoftmax(pooled, W, b.reshape(1, L), B, D, L)


# speedup vs baseline: 2.5055x; 2.2380x over previous
import functools

import jax
import jax.numpy as jnp
from jax import lax
from jax.experimental import pallas as pl
from jax.experimental.pallas import tpu as pltpu
from jax.experimental.pallas import tpu_sc as plsc

NC = 2
NS = 16
LANES = 16
NW = NC * NS

CH = 40
NP = 8


def _sc_gather_mean(input_ids, embed_table, B, S, D):
    BPW = B // NW
    DV = D // LANES
    NCH = S // CH

    mesh = plsc.VectorSubcoreMesh(core_axis_name="c", subcore_axis_name="s")

    @functools.partial(
        pl.kernel,
        out_type=jax.ShapeDtypeStruct((B, D), jnp.float32),
        mesh=mesh,
        scratch_types=[
            pltpu.VMEM((BPW * S,), jnp.int32),
            pltpu.VMEM((S, D), jnp.float32),
            pltpu.VMEM((S, D), jnp.float32),
            pltpu.VMEM((BPW, D), jnp.float32),
            pltpu.SemaphoreType.DMA,
            pltpu.SemaphoreType.DMA,
        ],
        compiler_params=pltpu.CompilerParams(use_tc_tiling_on_sc=False),
    )
    def sc_fn(idx_hbm, table_hbm, out_hbm, idx_v, buf_a, buf_b, out_v,
              sem_a, sem_b):
        wid = lax.axis_index("s") * NC + lax.axis_index("c")
        base = wid * BPW
        pltpu.sync_copy(idx_hbm.at[pl.ds(base * S, BPW * S)], idx_v)
        scale = jnp.float32(1.0 / S)

        def issue_row(r, buf, sem):
            for c in range(NCH):
                pltpu.async_copy(
                    table_hbm.at[idx_v.at[pl.ds(r * S + c * CH, CH)]],
                    buf.at[pl.ds(c * CH, CH)],
                    sem,
                )

        def drain_row(buf, sem):
            pltpu.make_async_copy(table_hbm.at[pl.ds(0, S)], buf, sem).wait()

        def compute_row(r, buf):
            for d in range(DV):
                p = [jnp.zeros((LANES,), jnp.float32) for _ in range(NP)]
                for j in range(S):
                    p[j % NP] = p[j % NP] + buf[j, pl.ds(d * LANES, LANES)]
                while len(p) > 1:
                    p = [p[i] + p[i + 1] for i in range(0, len(p), 2)]
                out_v[r, pl.ds(d * LANES, LANES)] = p[0] * scale

        issue_row(0, buf_a, sem_a)
        issue_row(1, buf_b, sem_b)

        def pair_fn(i, carry):
            r0 = 2 * i
            drain_row(buf_a, sem_a)
            compute_row(r0, buf_a)

            @pl.when(r0 + 2 < BPW)
            def _():
                issue_row(r0 + 2, buf_a, sem_a)

            drain_row(buf_b, sem_b)
            compute_row(r0 + 1, buf_b)

            @pl.when(r0 + 3 < BPW)
            def _():
                issue_row(r0 + 3, buf_b, sem_b)

            return carry

        lax.fori_loop(0, BPW // 2, pair_fn, 0)
        pltpu.sync_copy(out_v, out_hbm.at[pl.ds(base, BPW)])

    return sc_fn(input_ids, embed_table)


def _tc_transpose_pad(tt, V, D):
    BT = 2048
    grid = (V + BT - 1) // BT

    def body(t_ref, o_ref):
        t = t_ref[...].T.reshape(BT // 2, 2, D)
        o_ref[:, :D] = t[:, 0, :]
        o_ref[:, D:] = t[:, 1, :]

    return pl.pallas_call(
        body,
        grid=(grid,),
        in_specs=[pl.BlockSpec((D, BT), lambda i: (0, i))],
        out_specs=pl.BlockSpec((BT // 2, 2 * D), lambda i: (i, 0)),
        out_shape=jax.ShapeDtypeStruct((V // 2, 2 * D), jnp.float32),
    )(tt)


def _tc_linear_logsoftmax(x, W, b2, B, D, L):
    BT = 512

    def tc_body(x_ref, w_ref, b_ref, o_ref):
        logits = (
            jnp.dot(x_ref[...], w_ref[...], preferred_element_type=jnp.float32)
            + b_ref[...]
        )
        m = jnp.max(logits, axis=-1, keepdims=True)
        e = jnp.exp(logits - m)
        lse = jnp.log(jnp.sum(e, axis=-1, keepdims=True)) + m
        o_ref[...] = logits - lse

    return pl.pallas_call(
        tc_body,
        grid=(B // BT,),
        in_specs=[
            pl.BlockSpec((BT, D), lambda i: (i, 0)),
            pl.BlockSpec((D, L), lambda i: (0, 0)),
            pl.BlockSpec((1, L), lambda i: (0, 0)),
        ],
        out_specs=pl.BlockSpec((BT, L), lambda i: (i, 0)),
        out_shape=jax.ShapeDtypeStruct((B, L), jnp.float32),
    )(x, W, b2)


def kernel(input_ids, seq_len, embed_table, W, b):
    del seq_len
    B, S = input_ids.shape
    V, D = embed_table.shape
    L = W.shape[1]

    table_lin = _tc_transpose_pad(embed_table.T, V, D).reshape(V, D)
    idx_flat = input_ids.astype(jnp.int32).reshape(B * S)
    pooled = _sc_gather_mean(idx_flat, table_lin, B, S, D)
    return _tc_linear_logsoftmax(pooled, W, b.reshape(1, L), B, D, L)
